# baseline (speedup 1.0000x reference)
import functools

import jax
import jax.numpy as jnp
from jax import lax
from jax.experimental import pallas as pl
from jax.experimental.pallas import tpu as pltpu
from jax.experimental.pallas import tpu_sc as plsc

NN = 100000
EE = 3200000
CP = 16
KH = 10
NC, NS = 2, 16
NW = NC * NS
NPAD = 100352
STRIPE = NPAD // NS
WSTRIPE = NPAD // NW
CHUNK = 128
JC = 6
STEP = CHUNK * JC
STEPS = 132
EPT = STEP * STEPS
EPAD = EPT * NW
CC = 784

_MESH = plsc.VectorSubcoreMesh(core_axis_name="c", subcore_axis_name="s")
_SC_PARAMS = pltpu.CompilerParams(use_tc_tiling_on_sc=False)


def _edge_body(p_hbm, src_hbm, dst_hbm, zero_hbm, out_hbm,
               sidx, didx, rows, s_sh, semi, semg, sems):
    cid = lax.axis_index("c")
    sid = lax.axis_index("s")
    wid = cid * NS + sid
    base = sid * STRIPE
    @pl.when(cid == 0)
    def _():
        pltpu.sync_copy(p_hbm.at[pl.ds(base, STRIPE)],
                        s_sh.at[pl.ds(base, STRIPE)])

    @pl.when(cid != 0)
    def _():
        pltpu.sync_copy(zero_hbm.at[pl.ds(base, STRIPE)],
                        s_sh.at[pl.ds(base, STRIPE)])

    plsc.subcore_barrier()

    def idx_start(t, b):
        pltpu.async_copy(src_hbm.at[wid, t], sidx.at[b], semi)
        pltpu.async_copy(dst_hbm.at[wid, t], didx.at[b], semi)

    def idx_wait(b):
        pltpu.make_async_copy(src_hbm.at[wid, 0], sidx.at[b], semi).wait()
        pltpu.make_async_copy(dst_hbm.at[wid, 0], didx.at[b], semi).wait()

    def gath_start(b):
        for j in range(JC):
            pltpu.async_copy(p_hbm.at[sidx.at[b, j]], rows.at[b, j], semg)

    def gath_wait(b):
        for j in range(JC):
            pltpu.make_async_copy(p_hbm.at[sidx.at[b, j]], rows.at[b, j],
                                  semg).wait()

    def scat_start(b):
        for j in range(JC):
            pltpu.async_copy(rows.at[b, j], s_sh.at[didx.at[b, j]], sems,
                             add=True)

    def scat_wait(b):
        for j in range(JC):
            pltpu.make_async_copy(rows.at[b, j], s_sh.at[didx.at[b, j]],
                                  sems).wait()

    def do_step(t, b, wait_prev, prefetch):
        nb = 1 - b
        if wait_prev:
            scat_wait(nb)
        if prefetch:
            idx_start(t + 1, nb)
        gath_wait(b)
        scat_start(b)
        if prefetch:
            idx_wait(nb)
            gath_start(nb)

    idx_start(0, 0)
    idx_wait(0)
    gath_start(0)
    do_step(0, 0, False, True)

    def pair(tt, carry):
        do_step(2 * tt + 1, 1, True, True)
        do_step(2 * tt + 2, 0, True, True)
        return carry

    lax.fori_loop(0, (STEPS - 2) // 2, pair, 0)
    do_step(STEPS - 1, 1, True, False)
    scat_wait(1)

    plsc.subcore_barrier()
    pltpu.sync_copy(s_sh.at[pl.ds(base, STRIPE)],
                    out_hbm.at[cid, pl.ds(base, STRIPE)])


_edge_call = pl.kernel(
    _edge_body,
    out_type=jax.ShapeDtypeStruct((NC, NPAD, CP), jnp.float32),
    mesh=_MESH,
    scratch_types=[
        pltpu.VMEM((2, JC, CHUNK), jnp.int32),
        pltpu.VMEM((2, JC, CHUNK), jnp.int32),
        pltpu.VMEM((2, JC, CHUNK, CP), jnp.float32),
        pltpu.VMEM_SHARED((NPAD, CP), jnp.float32),
        pltpu.SemaphoreType.DMA,
        pltpu.SemaphoreType.DMA,
        pltpu.SemaphoreType.DMA,
    ],
    compiler_params=_SC_PARAMS,
)


CC2 = 448


def _edge2_body(s0_hbm, s1_hbm, w_hbm, q_hbm, src_hbm, dst_hbm, zero_hbm,
                out_hbm, pnew_hbm, s_sh, semc, semi, semg, sems):
    cid = lax.axis_index("c")
    sid = lax.axis_index("s")
    wid = cid * NS + sid
    base = sid * STRIPE

    def phase_a(b0, b1, bw, bq):
        for c in range(STRIPE // CC2):
            cb = base + c * CC2
            pltpu.async_copy(s0_hbm.at[pl.ds(cb, CC2)], b0, semc)
            pltpu.async_copy(s1_hbm.at[pl.ds(cb, CC2)], b1, semc)
            pltpu.async_copy(w_hbm.at[pl.ds(cb, CC2)], bw, semc)
            pltpu.async_copy(q_hbm.at[pl.ds(cb, CC2)], bq, semc)
            pltpu.make_async_copy(s0_hbm.at[pl.ds(cb, CC2)], b0, semc).wait()
            pltpu.make_async_copy(s1_hbm.at[pl.ds(cb, CC2)], b1, semc).wait()
            pltpu.make_async_copy(w_hbm.at[pl.ds(cb, CC2)], bw, semc).wait()
            pltpu.make_async_copy(q_hbm.at[pl.ds(cb, CC2)], bq, semc).wait()

            def row(r, carry):
                b0[r] = bw[r] * (b0[r] + b1[r]) + bq[r]
                return carry

            lax.fori_loop(0, CC2, row, 0)
            pltpu.sync_copy(b0, pnew_hbm.at[pl.ds(cb, CC2)])

    pl.run_scoped(
        phase_a,
        pltpu.VMEM((CC2, CP), jnp.float32),
        pltpu.VMEM((CC2, CP), jnp.float32),
        pltpu.VMEM((CC2, CP), jnp.float32),
        pltpu.VMEM((CC2, CP), jnp.float32),
    )

    @pl.when(cid == 0)
    def _():
        pltpu.sync_copy(pnew_hbm.at[pl.ds(base, STRIPE)],
                        s_sh.at[pl.ds(base, STRIPE)])

    @pl.when(cid != 0)
    def _():
        pltpu.sync_copy(zero_hbm.at[pl.ds(base, STRIPE)],
                        s_sh.at[pl.ds(base, STRIPE)])

    plsc.subcore_barrier()

    def phase_b(sidx, didx, rows):
        _edge_loop(pnew_hbm, src_hbm, dst_hbm, s_sh, sidx, didx, rows,
                   wid, semi, semg, sems)

    pl.run_scoped(
        phase_b,
        pltpu.VMEM((2, JC, CHUNK), jnp.int32),
        pltpu.VMEM((2, JC, CHUNK), jnp.int32),
        pltpu.VMEM((2, JC, CHUNK, CP), jnp.float32),
    )

    plsc.subcore_barrier()
    pltpu.sync_copy(s_sh.at[pl.ds(base, STRIPE)],
                    out_hbm.at[cid, pl.ds(base, STRIPE)])


def _edge_loop(p_hbm, src_hbm, dst_hbm, s_sh, sidx, didx, rows,
               wid, semi, semg, sems):
    def idx_start(t, b):
        pltpu.async_copy(src_hbm.at[wid, t], sidx.at[b], semi)
        pltpu.async_copy(dst_hbm.at[wid, t], didx.at[b], semi)

    def idx_wait(b):
        pltpu.make_async_copy(src_hbm.at[wid, 0], sidx.at[b], semi).wait()
        pltpu.make_async_copy(dst_hbm.at[wid, 0], didx.at[b], semi).wait()

    def gath_start(b):
        for j in range(JC):
            pltpu.async_copy(p_hbm.at[sidx.at[b, j]], rows.at[b, j], semg)

    def gath_wait(b):
        for j in range(JC):
            pltpu.make_async_copy(p_hbm.at[sidx.at[b, j]], rows.at[b, j],
                                  semg).wait()

    def scat_start(b):
        for j in range(JC):
            pltpu.async_copy(rows.at[b, j], s_sh.at[didx.at[b, j]], sems,
                             add=True)

    def scat_wait(b):
        for j in range(JC):
            pltpu.make_async_copy(rows.at[b, j], s_sh.at[didx.at[b, j]],
                                  sems).wait()

    def do_step(t, b, wait_prev, prefetch):
        nb = 1 - b
        if wait_prev:
            scat_wait(nb)
        if prefetch:
            idx_start(t + 1, nb)
        gath_wait(b)
        scat_start(b)
        if prefetch:
            idx_wait(nb)
            gath_start(nb)

    idx_start(0, 0)
    idx_wait(0)
    gath_start(0)
    do_step(0, 0, False, True)

    def pair(tt, carry):
        do_step(2 * tt + 1, 1, True, True)
        do_step(2 * tt + 2, 0, True, True)
        return carry

    lax.fori_loop(0, (STEPS - 2) // 2, pair, 0)
    do_step(STEPS - 1, 1, True, False)
    scat_wait(1)


_edge2_call = pl.kernel(
    _edge2_body,
    out_type=(jax.ShapeDtypeStruct((NC, NPAD, CP), jnp.float32),
              jax.ShapeDtypeStruct((NPAD, CP), jnp.float32)),
    mesh=_MESH,
    scratch_types=[
        pltpu.VMEM_SHARED((NPAD, CP), jnp.float32),
        pltpu.SemaphoreType.DMA,
        pltpu.SemaphoreType.DMA,
        pltpu.SemaphoreType.DMA,
        pltpu.SemaphoreType.DMA,
    ],
    compiler_params=_SC_PARAMS,
)


def _deg_body(dst_hbm, zero_hbm, out_hbm, didx, ones_v, s_sh, semi, sems):
    cid = lax.axis_index("c")
    sid = lax.axis_index("s")
    wid = cid * NS + sid
    base = sid * STRIPE

    def fill(i, carry):
        ones_v[i] = jnp.ones((CP,), jnp.float32)
        return carry

    lax.fori_loop(0, CHUNK, fill, 0)
    pltpu.sync_copy(zero_hbm.at[pl.ds(base, STRIPE)],
                    s_sh.at[pl.ds(base, STRIPE)])
    plsc.subcore_barrier()

    def idx_start(t, b):
        pltpu.async_copy(dst_hbm.at[wid, t], didx.at[b], semi)

    def idx_wait(b):
        pltpu.make_async_copy(dst_hbm.at[wid, 0], didx.at[b], semi).wait()

    def scat_start(b):
        for j in range(JC):
            pltpu.async_copy(ones_v, s_sh.at[didx.at[b, j]], sems, add=True)

    def scat_wait(b):
        for j in range(JC):
            pltpu.make_async_copy(ones_v, s_sh.at[didx.at[b, j]],
                                  sems).wait()

    def do_step(t, b, wait_prev, prefetch):
        nb = 1 - b
        if wait_prev:
            scat_wait(nb)
        if prefetch:
            idx_start(t + 1, nb)
        scat_start(b)
        if prefetch:
            idx_wait(nb)

    idx_start(0, 0)
    idx_wait(0)
    do_step(0, 0, False, True)

    def pair(tt, carry):
        do_step(2 * tt + 1, 1, True, True)
        do_step(2 * tt + 2, 0, True, True)
        return carry

    lax.fori_loop(0, (STEPS - 2) // 2, pair, 0)
    do_step(STEPS - 1, 1, True, False)
    scat_wait(1)

    plsc.subcore_barrier()
    pltpu.sync_copy(s_sh.at[pl.ds(base, STRIPE)],
                    out_hbm.at[cid, pl.ds(base, STRIPE)])


_deg_call = pl.kernel(
    _deg_body,
    out_type=jax.ShapeDtypeStruct((NC, NPAD, CP), jnp.float32),
    mesh=_MESH,
    scratch_types=[
        pltpu.VMEM((2, JC, CHUNK), jnp.int32),
        pltpu.VMEM((CHUNK, CP), jnp.float32),
        pltpu.VMEM_SHARED((NPAD, CP), jnp.float32),
        pltpu.SemaphoreType.DMA,
        pltpu.SemaphoreType.DMA,
    ],
    compiler_params=_SC_PARAMS,
)


def _combine_body(s0_hbm, s1_hbm, w_hbm, q_hbm, out_hbm,
                  b0, b1, bw, bq, bo):
    cid = lax.axis_index("c")
    sid = lax.axis_index("s")
    wid = cid * NS + sid
    for c in range(WSTRIPE // CC):
        base = wid * WSTRIPE + c * CC
        pltpu.sync_copy(s0_hbm.at[pl.ds(base, CC)], b0)
        pltpu.sync_copy(s1_hbm.at[pl.ds(base, CC)], b1)
        pltpu.sync_copy(w_hbm.at[pl.ds(base, CC)], bw)
        pltpu.sync_copy(q_hbm.at[pl.ds(base, CC)], bq)

        def row(r, carry):
            bo[r] = bw[r] * (b0[r] + b1[r]) + bq[r]
            return carry

        lax.fori_loop(0, CC, row, 0)
        pltpu.sync_copy(bo, out_hbm.at[pl.ds(base, CC)])


_combine_call = pl.kernel(
    _combine_body,
    out_type=jax.ShapeDtypeStruct((NPAD, CP), jnp.float32),
    mesh=_MESH,
    scratch_types=[
        pltpu.VMEM((CC, CP), jnp.float32),
        pltpu.VMEM((CC, CP), jnp.float32),
        pltpu.VMEM((CC, CP), jnp.float32),
        pltpu.VMEM((CC, CP), jnp.float32),
        pltpu.VMEM((CC, CP), jnp.float32),
    ],
    compiler_params=_SC_PARAMS,
)


_RB_MLP = 2000
_RB_PREP = 3136
_RB_OUT = 2000


def _mlp_body(x_ref, w1_ref, b1_ref, w2_ref, b2_ref, o_ref):
    h = jnp.dot(x_ref[...], w1_ref[...], preferred_element_type=jnp.float32)
    h = jnp.maximum(h + b1_ref[...], 0.0)
    o_ref[...] = (jnp.dot(h, w2_ref[...], preferred_element_type=jnp.float32)
                  + b2_ref[...])


def _mlp(x, W1, b1, W2p, b2p):
    grid = (NN // _RB_MLP,)
    return pl.pallas_call(
        _mlp_body,
        grid=grid,
        in_specs=[
            pl.BlockSpec((_RB_MLP, 128), lambda i: (i, 0)),
            pl.BlockSpec((128, 64), lambda i: (0, 0)),
            pl.BlockSpec((1, 64), lambda i: (0, 0)),
            pl.BlockSpec((64, CP), lambda i: (0, 0)),
            pl.BlockSpec((1, CP), lambda i: (0, 0)),
        ],
        out_specs=pl.BlockSpec((_RB_MLP, CP), lambda i: (i, 0)),
        out_shape=jax.ShapeDtypeStruct((NN, CP), jnp.float32),
    )(x, W1, b1, W2p, b2p)


def _prep_body(d0_ref, d1_ref, h0_ref, p_ref, w_ref, q_ref, sq_ref):
    deg = d0_ref[:, 0:1] + d1_ref[:, 0:1] + 1.0
    dinv = lax.rsqrt(deg)
    h0 = h0_ref[...]
    p_ref[...] = dinv * h0
    w_ref[...] = jnp.broadcast_to(0.9 * dinv * dinv, w_ref.shape)
    q_ref[...] = 0.1 * dinv * h0
    sq_ref[...] = jnp.broadcast_to(jnp.sqrt(deg), sq_ref.shape)


def _prep(d0, d1, h0p):
    grid = (NPAD // _RB_PREP,)
    sds = jax.ShapeDtypeStruct((NPAD, CP), jnp.float32)
    bs = pl.BlockSpec((_RB_PREP, CP), lambda i: (i, 0))
    return pl.pallas_call(
        _prep_body,
        grid=grid,
        in_specs=[bs, bs, bs],
        out_specs=[bs, bs, bs, bs],
        out_shape=[sds, sds, sds, sds],
    )(d0, d1, h0p)


def _out_body(s0_ref, s1_ref, w_ref, q_ref, sq_ref, o_ref):
    p = w_ref[...] * (s0_ref[...] + s1_ref[...]) + q_ref[...]
    h = p * sq_ref[...]
    col = lax.broadcasted_iota(jnp.int32, h.shape, 1)
    valid = col < 10
    hm = jnp.where(valid, h, -jnp.inf)
    m = jnp.max(hm, axis=1, keepdims=True)
    e = jnp.where(valid, jnp.exp(h - m), 0.0)
    s = jnp.sum(e, axis=1, keepdims=True)
    o_ref[...] = (h - m - jnp.log(s))[:, :10]


def _outk(s0, s1, w, q, sq):
    grid = (NN // _RB_OUT,)
    bs = pl.BlockSpec((_RB_OUT, CP), lambda i: (i, 0))
    return pl.pallas_call(
        _out_body,
        grid=grid,
        in_specs=[bs, bs, bs, bs, bs],
        out_specs=pl.BlockSpec((_RB_OUT, 10), lambda i: (i, 0)),
        out_shape=jax.ShapeDtypeStruct((NN, 10), jnp.float32),
    )(s0, s1, w, q, sq)


def kernel(x, edge_index, W1, b1, W2, b2):
    src = edge_index[0]
    dst = edge_index[1]
    padv = jnp.full((EPAD - EE,), NN, jnp.int32)
    srcp = jnp.concatenate([src, padv]).reshape(NW, STEPS, JC, CHUNK)
    dstp = jnp.concatenate([dst, padv]).reshape(NW, STEPS, JC, CHUNK)
    zeros16 = jnp.zeros((NPAD, CP), jnp.float32)

    W2p = jnp.pad(W2, ((0, 0), (0, CP - W2.shape[1])))
    b2p = jnp.pad(b2, (0, CP - b2.shape[0])).reshape(1, CP)
    b1r = b1.reshape(1, -1)

    degp = _deg_call(dstp, zeros16)
    h0 = _mlp(x, W1, b1r, W2p, b2p)
    h0p = jnp.pad(h0, ((0, NPAD - NN), (0, 0)))
    p, w16, q16, sq16 = _prep(degp[0], degp[1], h0p)

    for _ in range(KH - 1):
        sp = _edge_call(p, srcp, dstp, zeros16)
        p = _combine_call(sp[0], sp[1], w16, q16)
    sp = _edge_call(p, srcp, dstp, zeros16)

    return _outk(sp[0], sp[1], w16, q16, sq16)


# speedup vs baseline: 1.0042x; 1.0042x over previous
import functools

import jax
import jax.numpy as jnp
from jax import lax
from jax.experimental import pallas as pl
from jax.experimental.pallas import tpu as pltpu
from jax.experimental.pallas import tpu_sc as plsc

NN = 100000
EE = 3200000
CP = 16
KH = 10
NC, NS = 2, 16
NW = NC * NS
NPAD = 100352
STRIPE = NPAD // NS
WSTRIPE = NPAD // NW
CHUNK = 128
JC = 6
STEP = CHUNK * JC
STEPS = 132
EPT = STEP * STEPS
EPAD = EPT * NW
CC = 784

_MESH = plsc.VectorSubcoreMesh(core_axis_name="c", subcore_axis_name="s")
_SC_PARAMS = pltpu.CompilerParams(use_tc_tiling_on_sc=False)


def _edge_body(p_hbm, src_hbm, dst_hbm, zero_hbm, out_hbm,
               sidx, didx, rows, s_sh, semi, semg, sems):
    cid = lax.axis_index("c")
    sid = lax.axis_index("s")
    wid = cid * NS + sid
    base = sid * STRIPE
    @pl.when(cid == 0)
    def _():
        pltpu.sync_copy(p_hbm.at[pl.ds(base, STRIPE)],
                        s_sh.at[pl.ds(base, STRIPE)])

    @pl.when(cid != 0)
    def _():
        pltpu.sync_copy(zero_hbm.at[pl.ds(base, STRIPE)],
                        s_sh.at[pl.ds(base, STRIPE)])

    plsc.subcore_barrier()

    def idx_start(t, b):
        pltpu.async_copy(src_hbm.at[wid, t], sidx.at[b], semi)
        pltpu.async_copy(dst_hbm.at[wid, t], didx.at[b], semi)

    def idx_wait(b):
        pltpu.make_async_copy(src_hbm.at[wid, 0], sidx.at[b], semi).wait()
        pltpu.make_async_copy(dst_hbm.at[wid, 0], didx.at[b], semi).wait()

    def gath_start(b):
        for j in range(JC):
            pltpu.async_copy(p_hbm.at[sidx.at[b, j]], rows.at[b, j], semg)

    def gath_wait(b):
        for j in range(JC):
            pltpu.make_async_copy(p_hbm.at[sidx.at[b, j]], rows.at[b, j],
                                  semg).wait()

    def scat_start(b):
        for j in range(JC):
            pltpu.async_copy(rows.at[b, j], s_sh.at[didx.at[b, j]], sems,
                             add=True)

    def scat_wait(b):
        for j in range(JC):
            pltpu.make_async_copy(rows.at[b, j], s_sh.at[didx.at[b, j]],
                                  sems).wait()

    def do_step(t, b, wait_prev, prefetch):
        nb = 1 - b
        if wait_prev:
            scat_wait(nb)
        if prefetch:
            idx_start(t + 1, nb)
        gath_wait(b)
        scat_start(b)
        if prefetch:
            idx_wait(nb)
            gath_start(nb)

    idx_start(0, 0)
    idx_wait(0)
    gath_start(0)
    do_step(0, 0, False, True)

    def pair(tt, carry):
        do_step(2 * tt + 1, 1, True, True)
        do_step(2 * tt + 2, 0, True, True)
        return carry

    lax.fori_loop(0, (STEPS - 2) // 2, pair, 0)
    do_step(STEPS - 1, 1, True, False)
    scat_wait(1)

    plsc.subcore_barrier()
    pltpu.sync_copy(s_sh.at[pl.ds(base, STRIPE)],
                    out_hbm.at[cid, pl.ds(base, STRIPE)])


_edge_call = pl.kernel(
    _edge_body,
    out_type=jax.ShapeDtypeStruct((NC, NPAD, CP), jnp.float32),
    mesh=_MESH,
    scratch_types=[
        pltpu.VMEM((2, JC, CHUNK), jnp.int32),
        pltpu.VMEM((2, JC, CHUNK), jnp.int32),
        pltpu.VMEM((2, JC, CHUNK, CP), jnp.float32),
        pltpu.VMEM_SHARED((NPAD, CP), jnp.float32),
        pltpu.SemaphoreType.DMA,
        pltpu.SemaphoreType.DMA,
        pltpu.SemaphoreType.DMA,
    ],
    compiler_params=_SC_PARAMS,
)


CC2 = 448


def _edge2_body(s0_hbm, s1_hbm, w_hbm, q_hbm, src_hbm, dst_hbm, zero_hbm,
                out_hbm, pnew_hbm, s_sh, semc, semi, semg, sems):
    cid = lax.axis_index("c")
    sid = lax.axis_index("s")
    wid = cid * NS + sid
    base = sid * STRIPE

    def phase_a(b0, b1, bw, bq):
        for c in range(STRIPE // CC2):
            cb = base + c * CC2
            pltpu.async_copy(s0_hbm.at[pl.ds(cb, CC2)], b0, semc)
            pltpu.async_copy(s1_hbm.at[pl.ds(cb, CC2)], b1, semc)
            pltpu.async_copy(w_hbm.at[pl.ds(cb, CC2)], bw, semc)
            pltpu.async_copy(q_hbm.at[pl.ds(cb, CC2)], bq, semc)
            pltpu.make_async_copy(s0_hbm.at[pl.ds(cb, CC2)], b0, semc).wait()
            pltpu.make_async_copy(s1_hbm.at[pl.ds(cb, CC2)], b1, semc).wait()
            pltpu.make_async_copy(w_hbm.at[pl.ds(cb, CC2)], bw, semc).wait()
            pltpu.make_async_copy(q_hbm.at[pl.ds(cb, CC2)], bq, semc).wait()

            def row(r, carry):
                b0[r] = bw[r] * (b0[r] + b1[r]) + bq[r]
                return carry

            lax.fori_loop(0, CC2, row, 0)
            pltpu.sync_copy(b0, pnew_hbm.at[pl.ds(cb, CC2)])

    pl.run_scoped(
        phase_a,
        pltpu.VMEM((CC2, CP), jnp.float32),
        pltpu.VMEM((CC2, CP), jnp.float32),
        pltpu.VMEM((CC2, CP), jnp.float32),
        pltpu.VMEM((CC2, CP), jnp.float32),
    )

    @pl.when(cid == 0)
    def _():
        pltpu.sync_copy(pnew_hbm.at[pl.ds(base, STRIPE)],
                        s_sh.at[pl.ds(base, STRIPE)])

    @pl.when(cid != 0)
    def _():
        pltpu.sync_copy(zero_hbm.at[pl.ds(base, STRIPE)],
                        s_sh.at[pl.ds(base, STRIPE)])

    plsc.subcore_barrier()

    def phase_b(sidx, didx, rows):
        _edge_loop(pnew_hbm, src_hbm, dst_hbm, s_sh, sidx, didx, rows,
                   wid, semi, semg, sems)

    pl.run_scoped(
        phase_b,
        pltpu.VMEM((2, JC, CHUNK), jnp.int32),
        pltpu.VMEM((2, JC, CHUNK), jnp.int32),
        pltpu.VMEM((2, JC, CHUNK, CP), jnp.float32),
    )

    plsc.subcore_barrier()
    pltpu.sync_copy(s_sh.at[pl.ds(base, STRIPE)],
                    out_hbm.at[cid, pl.ds(base, STRIPE)])


def _edge_loop(p_hbm, src_hbm, dst_hbm, s_sh, sidx, didx, rows,
               wid, semi, semg, sems):
    def idx_start(t, b):
        pltpu.async_copy(src_hbm.at[wid, t], sidx.at[b], semi)
        pltpu.async_copy(dst_hbm.at[wid, t], didx.at[b], semi)

    def idx_wait(b):
        pltpu.make_async_copy(src_hbm.at[wid, 0], sidx.at[b], semi).wait()
        pltpu.make_async_copy(dst_hbm.at[wid, 0], didx.at[b], semi).wait()

    def gath_start(b):
        for j in range(JC):
            pltpu.async_copy(p_hbm.at[sidx.at[b, j]], rows.at[b, j], semg)

    def gath_wait(b):
        for j in range(JC):
            pltpu.make_async_copy(p_hbm.at[sidx.at[b, j]], rows.at[b, j],
                                  semg).wait()

    def scat_start(b):
        for j in range(JC):
            pltpu.async_copy(rows.at[b, j], s_sh.at[didx.at[b, j]], sems,
                             add=True)

    def scat_wait(b):
        for j in range(JC):
            pltpu.make_async_copy(rows.at[b, j], s_sh.at[didx.at[b, j]],
                                  sems).wait()

    def do_step(t, b, wait_prev, prefetch):
        nb = 1 - b
        if wait_prev:
            scat_wait(nb)
        if prefetch:
            idx_start(t + 1, nb)
        gath_wait(b)
        scat_start(b)
        if prefetch:
            idx_wait(nb)
            gath_start(nb)

    idx_start(0, 0)
    idx_wait(0)
    gath_start(0)
    do_step(0, 0, False, True)

    def pair(tt, carry):
        do_step(2 * tt + 1, 1, True, True)
        do_step(2 * tt + 2, 0, True, True)
        return carry

    lax.fori_loop(0, (STEPS - 2) // 2, pair, 0)
    do_step(STEPS - 1, 1, True, False)
    scat_wait(1)


_edge2_call = pl.kernel(
    _edge2_body,
    out_type=(jax.ShapeDtypeStruct((NC, NPAD, CP), jnp.float32),
              jax.ShapeDtypeStruct((NPAD, CP), jnp.float32)),
    mesh=_MESH,
    scratch_types=[
        pltpu.VMEM_SHARED((NPAD, CP), jnp.float32),
        pltpu.SemaphoreType.DMA,
        pltpu.SemaphoreType.DMA,
        pltpu.SemaphoreType.DMA,
        pltpu.SemaphoreType.DMA,
    ],
    compiler_params=_SC_PARAMS,
)


def _deg_body(dst_hbm, zero_hbm, out_hbm, didx, ones_v, s_sh, semi, sems):
    cid = lax.axis_index("c")
    sid = lax.axis_index("s")
    wid = cid * NS + sid
    base = sid * STRIPE

    def fill(i, carry):
        ones_v[i] = jnp.ones((CP,), jnp.float32)
        return carry

    lax.fori_loop(0, CHUNK, fill, 0)
    pltpu.sync_copy(zero_hbm.at[pl.ds(base, STRIPE)],
                    s_sh.at[pl.ds(base, STRIPE)])
    plsc.subcore_barrier()

    def idx_start(t, b):
        pltpu.async_copy(dst_hbm.at[wid, t], didx.at[b], semi)

    def idx_wait(b):
        pltpu.make_async_copy(dst_hbm.at[wid, 0], didx.at[b], semi).wait()

    def scat_start(b):
        for j in range(JC):
            pltpu.async_copy(ones_v, s_sh.at[didx.at[b, j]], sems, add=True)

    def scat_wait(b):
        for j in range(JC):
            pltpu.make_async_copy(ones_v, s_sh.at[didx.at[b, j]],
                                  sems).wait()

    def do_step(t, b, wait_prev, prefetch):
        nb = 1 - b
        if wait_prev:
            scat_wait(nb)
        if prefetch:
            idx_start(t + 1, nb)
        scat_start(b)
        if prefetch:
            idx_wait(nb)

    idx_start(0, 0)
    idx_wait(0)
    do_step(0, 0, False, True)

    def pair(tt, carry):
        do_step(2 * tt + 1, 1, True, True)
        do_step(2 * tt + 2, 0, True, True)
        return carry

    lax.fori_loop(0, (STEPS - 2) // 2, pair, 0)
    do_step(STEPS - 1, 1, True, False)
    scat_wait(1)

    plsc.subcore_barrier()
    pltpu.sync_copy(s_sh.at[pl.ds(base, STRIPE)],
                    out_hbm.at[cid, pl.ds(base, STRIPE)])


_deg_call = pl.kernel(
    _deg_body,
    out_type=jax.ShapeDtypeStruct((NC, NPAD, CP), jnp.float32),
    mesh=_MESH,
    scratch_types=[
        pltpu.VMEM((2, JC, CHUNK), jnp.int32),
        pltpu.VMEM((CHUNK, CP), jnp.float32),
        pltpu.VMEM_SHARED((NPAD, CP), jnp.float32),
        pltpu.SemaphoreType.DMA,
        pltpu.SemaphoreType.DMA,
    ],
    compiler_params=_SC_PARAMS,
)


def _combine_body(s0_hbm, s1_hbm, w_hbm, q_hbm, out_hbm,
                  b0, b1, bw, bq, sem0, sem1):
    cid = lax.axis_index("c")
    sid = lax.axis_index("s")
    wid = cid * NS + sid
    sems = (sem0, sem1)
    nch = WSTRIPE // CC

    def ld(c, s):
        base = wid * WSTRIPE + c * CC
        pltpu.async_copy(s0_hbm.at[pl.ds(base, CC)], b0.at[s], sems[s])
        pltpu.async_copy(s1_hbm.at[pl.ds(base, CC)], b1.at[s], sems[s])
        pltpu.async_copy(w_hbm.at[pl.ds(base, CC)], bw.at[s], sems[s])
        pltpu.async_copy(q_hbm.at[pl.ds(base, CC)], bq.at[s], sems[s])

    def drain(c, s):
        base = wid * WSTRIPE + c * CC
        pltpu.make_async_copy(s0_hbm.at[pl.ds(base, CC)], b0.at[s],
                              sems[s]).wait()
        pltpu.make_async_copy(s1_hbm.at[pl.ds(base, CC)], b1.at[s],
                              sems[s]).wait()
        pltpu.make_async_copy(w_hbm.at[pl.ds(base, CC)], bw.at[s],
                              sems[s]).wait()
        pltpu.make_async_copy(q_hbm.at[pl.ds(base, CC)], bq.at[s],
                              sems[s]).wait()

    ld(0, 0)
    ld(1, 1)
    for c in range(nch):
        s = c % 2
        base = wid * WSTRIPE + c * CC
        drain(c, s)

        def row(r, carry):
            b0[s, r] = bw[s, r] * (b0[s, r] + b1[s, r]) + bq[s, r]
            return carry

        lax.fori_loop(0, CC, row, 0)
        pltpu.sync_copy(b0.at[s], out_hbm.at[pl.ds(base, CC)])
        if c + 2 < nch:
            ld(c + 2, s)


_combine_call = pl.kernel(
    _combine_body,
    out_type=jax.ShapeDtypeStruct((NPAD, CP), jnp.float32),
    mesh=_MESH,
    scratch_types=[
        pltpu.VMEM((2, CC, CP), jnp.float32),
        pltpu.VMEM((2, CC, CP), jnp.float32),
        pltpu.VMEM((2, CC, CP), jnp.float32),
        pltpu.VMEM((2, CC, CP), jnp.float32),
        pltpu.SemaphoreType.DMA,
        pltpu.SemaphoreType.DMA,
    ],
    compiler_params=_SC_PARAMS,
)


_RB_MLP = 2000
_RB_PREP = 3136
_RB_OUT = 2000


def _mlp_body(x_ref, w1_ref, b1_ref, w2_ref, b2_ref, o_ref):
    h = jnp.dot(x_ref[...], w1_ref[...], preferred_element_type=jnp.float32)
    h = jnp.maximum(h + b1_ref[...], 0.0)
    o_ref[...] = (jnp.dot(h, w2_ref[...], preferred_element_type=jnp.float32)
                  + b2_ref[...])


def _mlp(x, W1, b1, W2p, b2p):
    grid = (NN // _RB_MLP,)
    return pl.pallas_call(
        _mlp_body,
        grid=grid,
        in_specs=[
            pl.BlockSpec((_RB_MLP, 128), lambda i: (i, 0)),
            pl.BlockSpec((128, 64), lambda i: (0, 0)),
            pl.BlockSpec((1, 64), lambda i: (0, 0)),
            pl.BlockSpec((64, CP), lambda i: (0, 0)),
            pl.BlockSpec((1, CP), lambda i: (0, 0)),
        ],
        out_specs=pl.BlockSpec((_RB_MLP, CP), lambda i: (i, 0)),
        out_shape=jax.ShapeDtypeStruct((NN, CP), jnp.float32),
    )(x, W1, b1, W2p, b2p)


def _prep_body(d0_ref, d1_ref, h0_ref, p_ref, w_ref, q_ref, sq_ref):
    deg = d0_ref[:, 0:1] + d1_ref[:, 0:1] + 1.0
    dinv = lax.rsqrt(deg)
    h0 = h0_ref[...]
    p_ref[...] = dinv * h0
    w_ref[...] = jnp.broadcast_to(0.9 * dinv * dinv, w_ref.shape)
    q_ref[...] = 0.1 * dinv * h0
    sq_ref[...] = jnp.broadcast_to(jnp.sqrt(deg), sq_ref.shape)


def _prep(d0, d1, h0p):
    grid = (NPAD // _RB_PREP,)
    sds = jax.ShapeDtypeStruct((NPAD, CP), jnp.float32)
    bs = pl.BlockSpec((_RB_PREP, CP), lambda i: (i, 0))
    return pl.pallas_call(
        _prep_body,
        grid=grid,
        in_specs=[bs, bs, bs],
        out_specs=[bs, bs, bs, bs],
        out_shape=[sds, sds, sds, sds],
    )(d0, d1, h0p)


def _out_body(s0_ref, s1_ref, w_ref, q_ref, sq_ref, o_ref):
    p = w_ref[...] * (s0_ref[...] + s1_ref[...]) + q_ref[...]
    h = p * sq_ref[...]
    col = lax.broadcasted_iota(jnp.int32, h.shape, 1)
    valid = col < 10
    hm = jnp.where(valid, h, -jnp.inf)
    m = jnp.max(hm, axis=1, keepdims=True)
    e = jnp.where(valid, jnp.exp(h - m), 0.0)
    s = jnp.sum(e, axis=1, keepdims=True)
    o_ref[...] = (h - m - jnp.log(s))[:, :10]


def _outk(s0, s1, w, q, sq):
    grid = (NN // _RB_OUT,)
    bs = pl.BlockSpec((_RB_OUT, CP), lambda i: (i, 0))
    return pl.pallas_call(
        _out_body,
        grid=grid,
        in_specs=[bs, bs, bs, bs, bs],
        out_specs=pl.BlockSpec((_RB_OUT, 10), lambda i: (i, 0)),
        out_shape=jax.ShapeDtypeStruct((NN, 10), jnp.float32),
    )(s0, s1, w, q, sq)


def kernel(x, edge_index, W1, b1, W2, b2):
    src = edge_index[0]
    dst = edge_index[1]
    padv = jnp.full((EPAD - EE,), NN, jnp.int32)
    srcp = jnp.concatenate([src, padv]).reshape(NW, STEPS, JC, CHUNK)
    dstp = jnp.concatenate([dst, padv]).reshape(NW, STEPS, JC, CHUNK)
    zeros16 = jnp.zeros((NPAD, CP), jnp.float32)

    W2p = jnp.pad(W2, ((0, 0), (0, CP - W2.shape[1])))
    b2p = jnp.pad(b2, (0, CP - b2.shape[0])).reshape(1, CP)
    b1r = b1.reshape(1, -1)

    degp = _deg_call(dstp, zeros16)
    h0 = _mlp(x, W1, b1r, W2p, b2p)
    h0p = jnp.pad(h0, ((0, NPAD - NN), (0, 0)))
    p, w16, q16, sq16 = _prep(degp[0], degp[1], h0p)

    for _ in range(KH - 1):
        sp = _edge_call(p, srcp, dstp, zeros16)
        p = _combine_call(sp[0], sp[1], w16, q16)
    sp = _edge_call(p, srcp, dstp, zeros16)

    return _outk(sp[0], sp[1], w16, q16, sq16)
